# baseline (speedup 1.0000x reference)
import functools
import jax
import jax.numpy as jnp
from jax import lax
from jax.experimental import pallas as pl
from jax.experimental.pallas import tpu as pltpu
from jax.experimental.pallas import tpu_sc as plsc

DIM = 64
ENCDIM = 64
OUTD = DIM + ENCDIM
BATCH = 16384
VOC = 1000000

_NC = 2
_NS = 16
_NW = _NC * _NS
_BPW = BATCH // _NW
_L = 16
_CSZ = 32
_NCH = _BPW // _CSZ

_mesh = plsc.VectorSubcoreMesh(core_axis_name="c", subcore_axis_name="s")


@functools.partial(
    pl.kernel,
    mesh=_mesh,
    out_type=jax.ShapeDtypeStruct((BATCH,), jnp.int32),
    scratch_types=[
        pltpu.VMEM((_BPW,), jnp.int32),
        pltpu.VMEM((_BPW,), jnp.int32),
        pltpu.SemaphoreType.DMA,
    ],
)
def _remap_kernel(ids_hbm, map_hbm, mid_hbm, ids_v, mid_v, sem):
    wid = lax.axis_index("s") * _NC + lax.axis_index("c")
    base = pl.multiple_of(wid * _BPW, _BPW)
    pltpu.sync_copy(ids_hbm.at[pl.ds(base, _BPW)], ids_v)
    pltpu.async_copy(map_hbm.at[ids_v], mid_v, sem).wait()
    pltpu.sync_copy(mid_v, mid_hbm.at[pl.ds(base, _BPW)])


@functools.partial(
    pl.kernel,
    mesh=_mesh,
    out_type=jax.ShapeDtypeStruct((BATCH, OUTD), jnp.float32),
    scratch_types=[
        pltpu.VMEM((_BPW,), jnp.int32),
        pltpu.VMEM((2, _CSZ, 8, DIM), jnp.float32),
        pltpu.VMEM((_BPW // 2, 2 * ENCDIM), jnp.float32),
        pltpu.VMEM((2, _CSZ, OUTD), jnp.float32),
        pltpu.SemaphoreType.DMA,
        pltpu.SemaphoreType.DMA,
        pltpu.SemaphoreType.DMA,
        pltpu.SemaphoreType.DMA,
        pltpu.SemaphoreType.DMA,
    ],
)
def _dil_kernel(mids_hbm, prev2_hbm, emb_hbm, out_hbm,
                mid_v, grp_v, prev_v, out_v, sem0, sem1, semp,
                semo0, semo1):
    wid = lax.axis_index("s") * _NC + lax.axis_index("c")
    base = pl.multiple_of(wid * _BPW, _BPW)
    hbase = pl.multiple_of(wid * (_BPW // 2), _BPW // 2)
    prev_cp = pltpu.async_copy(prev2_hbm.at[pl.ds(hbase, _BPW // 2)],
                               prev_v, semp)
    pltpu.sync_copy(mids_hbm.at[pl.ds(base, _BPW)], mid_v)

    sems = [sem0, sem1]
    semo = [semo0, semo1]
    emb3 = emb_hbm.reshape(VOC // 8, 8, DIM)

    def _fire(q, b):
        for j in range(_CSZ // _L):
            m16 = mid_v[pl.ds(q * _CSZ + _L * j, _L)]
            for r2 in range(_L):
                g8 = pl.multiple_of((m16[r2] >> 3) * 8, 8)
                pltpu.async_copy(emb_hbm.at[pl.ds(g8, 8)],
                                 grp_v.at[b, _L * j + r2], sems[b])

    _fire(0, 0)
    prev_cp.wait()

    def _do_chunk(q, b):
        pltpu.make_async_copy(emb3.at[pl.ds(0, _CSZ)], grp_v.at[b],
                              sems[b]).wait()

        @pl.when(q >= 2)
        def _():
            pltpu.make_async_copy(out_v.at[b],
                                  out_hbm.at[pl.ds(base, _CSZ)],
                                  semo[b]).wait()

        def _grp(j, c2):
            m16 = mid_v[pl.ds(q * _CSZ + _L * j, _L)]
            for r2 in range(_L):
                s = m16[r2] & 7
                r = _L * j + r2
                for k in range(DIM // _L):
                    out_v[b, r, pl.ds(_L * k, _L)] = grp_v[b, r, s,
                                                           pl.ds(_L * k, _L)]
                poff = (r2 & 1) * ENCDIM
                prow = q * (_CSZ // 2) + (_L // 2) * j + (r2 >> 1)
                for k in range(ENCDIM // _L):
                    out_v[b, r, pl.ds(DIM + _L * k, _L)] = prev_v[
                        prow, pl.ds(poff + _L * k, _L)]
            return c2

        lax.fori_loop(0, _CSZ // _L, _grp, 0)
        pltpu.async_copy(out_v.at[b], out_hbm.at[
            pl.ds(pl.multiple_of(base + q * _CSZ, _CSZ), _CSZ)], semo[b])

    def _pair(q2, carry):
        c0 = q2 * 2
        _fire(c0 + 1, 1)
        _do_chunk(c0, 0)

        @pl.when(c0 + 2 < _NCH)
        def _():
            _fire(c0 + 2, 0)

        _do_chunk(c0 + 1, 1)
        return carry

    lax.fori_loop(0, _NCH // 2, _pair, 0)
    for b in range(2):
        pltpu.make_async_copy(out_v.at[b], out_hbm.at[pl.ds(base, _CSZ)],
                              semo[b]).wait()


def kernel(ids, prev_inp_summ, emb_table, mapper):
    prev2 = prev_inp_summ.reshape(BATCH // 2, 2 * ENCDIM)
    head = lax.optimization_barrier(lax.slice(emb_table, (0, 0), (8, DIM)))
    emb2 = lax.dynamic_update_slice(emb_table, head, (0, 0))
    mids = _remap_kernel(ids.astype(jnp.int32), mapper.astype(jnp.int32))
    return _dil_kernel(mids, prev2, emb2)


# speedup vs baseline: 1.0174x; 1.0174x over previous
import functools
import jax
import jax.numpy as jnp
from jax import lax
from jax.experimental import pallas as pl
from jax.experimental.pallas import tpu as pltpu
from jax.experimental.pallas import tpu_sc as plsc

DIM = 64
ENCDIM = 64
OUTD = DIM + ENCDIM
BATCH = 16384
VOC = 1000000

_NC = 2
_NS = 16
_NW = _NC * _NS
_BPW = BATCH // _NW
_L = 16
_CSZ = 32
_NCH = _BPW // _CSZ

_mesh = plsc.VectorSubcoreMesh(core_axis_name="c", subcore_axis_name="s")


@functools.partial(
    pl.kernel,
    mesh=_mesh,
    out_type=jax.ShapeDtypeStruct((BATCH, OUTD), jnp.float32),
    scratch_types=[
        pltpu.VMEM((_BPW,), jnp.int32),
        pltpu.VMEM((_BPW,), jnp.int32),
        pltpu.VMEM((2, _CSZ, 8, DIM), jnp.float32),
        pltpu.VMEM((_BPW // 2, 2 * ENCDIM), jnp.float32),
        pltpu.VMEM((2, _CSZ, OUTD), jnp.float32),
        pltpu.SemaphoreType.DMA,
        pltpu.SemaphoreType.DMA,
        pltpu.SemaphoreType.DMA,
        pltpu.SemaphoreType.DMA,
        pltpu.SemaphoreType.DMA,
    ],
)
def _dil_kernel(ids_hbm, prev2_hbm, emb_hbm, map_hbm, out_hbm,
                ids_v, mid_v, grp_v, prev_v, out_v, sem0, sem1, semp,
                semo0, semo1):
    wid = lax.axis_index("s") * _NC + lax.axis_index("c")
    base = pl.multiple_of(wid * _BPW, _BPW)
    hbase = pl.multiple_of(wid * (_BPW // 2), _BPW // 2)
    prev_cp = pltpu.async_copy(prev2_hbm.at[pl.ds(hbase, _BPW // 2)],
                               prev_v, semp)
    pltpu.sync_copy(ids_hbm.at[pl.ds(base, _BPW)], ids_v)
    pltpu.async_copy(map_hbm.at[ids_v], mid_v, sem0).wait()

    sems = [sem0, sem1]
    semo = [semo0, semo1]
    emb3 = emb_hbm.reshape(VOC // 8, 8, DIM)

    def _fire(q, b):
        for j in range(_CSZ // _L):
            m16 = mid_v[pl.ds(q * _CSZ + _L * j, _L)]
            for r2 in range(_L):
                g8 = pl.multiple_of((m16[r2] >> 3) * 8, 8)
                pltpu.async_copy(emb_hbm.at[pl.ds(g8, 8)],
                                 grp_v.at[b, _L * j + r2], sems[b])

    _fire(0, 0)
    prev_cp.wait()

    def _do_chunk(q, b):
        pltpu.make_async_copy(emb3.at[pl.ds(0, _CSZ)], grp_v.at[b],
                              sems[b]).wait()

        @pl.when(q >= 2)
        def _():
            pltpu.make_async_copy(out_v.at[b],
                                  out_hbm.at[pl.ds(base, _CSZ)],
                                  semo[b]).wait()

        def _grp(j, c2):
            m16 = mid_v[pl.ds(q * _CSZ + _L * j, _L)]
            for r2 in range(_L):
                s = m16[r2] & 7
                r = _L * j + r2
                for k in range(DIM // _L):
                    out_v[b, r, pl.ds(_L * k, _L)] = grp_v[b, r, s,
                                                           pl.ds(_L * k, _L)]
                poff = (r2 & 1) * ENCDIM
                prow = q * (_CSZ // 2) + (_L // 2) * j + (r2 >> 1)
                for k in range(ENCDIM // _L):
                    out_v[b, r, pl.ds(DIM + _L * k, _L)] = prev_v[
                        prow, pl.ds(poff + _L * k, _L)]
            return c2

        lax.fori_loop(0, _CSZ // _L, _grp, 0)
        pltpu.async_copy(out_v.at[b], out_hbm.at[
            pl.ds(pl.multiple_of(base + q * _CSZ, _CSZ), _CSZ)], semo[b])

    def _pair(q2, carry):
        c0 = q2 * 2
        _fire(c0 + 1, 1)
        _do_chunk(c0, 0)

        @pl.when(c0 + 2 < _NCH)
        def _():
            _fire(c0 + 2, 0)

        _do_chunk(c0 + 1, 1)
        return carry

    lax.fori_loop(0, _NCH // 2, _pair, 0)
    for b in range(2):
        pltpu.make_async_copy(out_v.at[b], out_hbm.at[pl.ds(base, _CSZ)],
                              semo[b]).wait()


def kernel(ids, prev_inp_summ, emb_table, mapper):
    prev2 = prev_inp_summ.reshape(BATCH // 2, 2 * ENCDIM)
    head = lax.optimization_barrier(lax.slice(emb_table, (0, 0), (8, DIM)))
    emb2 = lax.dynamic_update_slice(emb_table, head, (0, 0))
    return _dil_kernel(ids.astype(jnp.int32), prev2, emb2,
                       mapper.astype(jnp.int32))
